# baseline (speedup 1.0000x reference)
import jax
import jax.numpy as jnp
from jax.experimental import pallas as pl
from jax.experimental.pallas import tpu as pltpu

BATCH = 16384
NF = 32
HID = 128
BM = 2048


def _mlp_body(ue_ref, me_ref, w1u_ref, w1m_ref, b1_ref,
              w2_ref, b2_ref, wf_ref, bf_ref, out_ref):
    x = ue_ref[...] @ w1u_ref[...] + me_ref[...] @ w1m_ref[...] + b1_ref[...]
    x = jnp.maximum(x, 0.0)
    x = jnp.maximum(x @ w2_ref[...] + b2_ref[...], 0.0)
    z = jnp.sum(x * wf_ref[...], axis=1, keepdims=True) + bf_ref[0, 0]
    out_ref[...] = jax.nn.sigmoid(z) * 4.0 + 1.0


def _mlp(ue, me, W1, b1, W2, b2, Wf, bf):
    grid = (BATCH // BM,)
    w1u = W1[:NF]
    w1m = W1[NF:]
    b1r = b1.reshape(1, HID)
    b2r = b2.reshape(1, HID)
    wfr = Wf.reshape(1, HID)
    bfr = bf.reshape(1, 1)
    full = lambda shape: pl.BlockSpec(shape, lambda i: (0, 0))
    return pl.pallas_call(
        _mlp_body,
        grid=grid,
        in_specs=[
            pl.BlockSpec((BM, NF), lambda i: (i, 0)),
            pl.BlockSpec((BM, NF), lambda i: (i, 0)),
            full((NF, HID)),
            full((NF, HID)),
            full((1, HID)),
            full((HID, HID)),
            full((1, HID)),
            full((1, HID)),
            full((1, 1)),
        ],
        out_specs=pl.BlockSpec((BM, 1), lambda i: (i, 0)),
        out_shape=jax.ShapeDtypeStruct((BATCH, 1), jnp.float32),
        compiler_params=pltpu.CompilerParams(
            dimension_semantics=("arbitrary",)),
    )(ue, me, w1u, w1m, b1r, W2, b2r, wfr, bfr)


def kernel(users, movies, u_table, m_table, W1, b1, W2, b2, Wf, bf):
    h = BATCH // 2
    ue0 = jnp.take(u_table, users[:h], axis=0, mode="clip")
    me0 = jnp.take(m_table, movies[:h], axis=0, mode="clip")
    ue1 = jnp.take(u_table, users[h:], axis=0, mode="clip")
    me1 = jnp.take(m_table, movies[h:], axis=0, mode="clip")
    ue = jnp.concatenate([ue0, ue1], axis=0)
    me = jnp.concatenate([me0, me1], axis=0)
    return _mlp(ue, me, W1, b1, W2, b2, Wf, bf)
